# baseline (speedup 1.0000x reference)
import functools

import jax
import jax.numpy as jnp
from jax.experimental import pallas as pl

N_MOD = 4
BATCH = 16384
D_IN = 64
EMBRACE = 64
BB = 2048

_K0 = 0
_K1 = 42
_K2 = _K0 ^ _K1 ^ 0x1BD11BDA
_KS = (_K0, _K1, _K2)
_ROT = ((13, 15, 26, 6), (17, 29, 16, 24))


def _threefry_bits(x1_i32):
    x0 = jnp.zeros_like(x1_i32)
    x1 = x1_i32
    x0 = x0 + jnp.int32(_KS[0])
    x1 = x1 + jnp.int32(_KS[1])
    for i in range(5):
        for r in _ROT[i % 2]:
            x0 = x0 + x1
            x1 = (x1 << r) | jax.lax.shift_right_logical(x1, 32 - r)
            x1 = x1 ^ x0
        x0 = x0 + jnp.int32(jnp.uint32(_KS[(i + 1) % 3]).astype(jnp.int32))
        x1 = x1 + jnp.int32((jnp.uint32(_KS[(i + 2) % 3]) + jnp.uint32(i + 1)).astype(jnp.int32))
    return x0 ^ x1


def _fuse_kernel(x_ref, w_ref, b_ref, o_ref):
    docked = []
    for m in range(N_MOD):
        d = jnp.dot(x_ref[m], w_ref[m], preferred_element_type=jnp.float32)
        docked.append(jnp.maximum(d + b_ref[m][None, :], 0.0))

    row0 = pl.program_id(0) * BB
    rows = jax.lax.broadcasted_iota(jnp.int32, (BB, EMBRACE), 0) + row0
    cols = jax.lax.broadcasted_iota(jnp.int32, (BB, EMBRACE), 1)
    base = rows * (EMBRACE * N_MOD) + cols * N_MOD

    best = jax.lax.shift_right_logical(_threefry_bits(base), 9)
    res = docked[0]
    for m in range(1, N_MOD):
        v = jax.lax.shift_right_logical(_threefry_bits(base + m), 9)
        take = v > best
        best = jnp.where(take, v, best)
        res = jnp.where(take, docked[m], res)
    o_ref[...] = res


@jax.jit
def kernel(outputs1, outputs2, available, W, b):
    del outputs2, available
    grid = (BATCH // BB,)
    return pl.pallas_call(
        _fuse_kernel,
        grid=grid,
        in_specs=[
            pl.BlockSpec((N_MOD, BB, D_IN), lambda i: (0, i, 0)),
            pl.BlockSpec((N_MOD, D_IN, EMBRACE), lambda i: (0, 0, 0)),
            pl.BlockSpec((N_MOD, EMBRACE), lambda i: (0, 0)),
        ],
        out_specs=pl.BlockSpec((BB, EMBRACE), lambda i: (i, 0)),
        out_shape=jax.ShapeDtypeStruct((BATCH, EMBRACE), jnp.float32),
    )(outputs1, W, b)


# speedup vs baseline: 1.4484x; 1.4484x over previous
import jax
import jax.numpy as jnp
from jax.experimental import pallas as pl

N_MOD = 4
BATCH = 16384
D_IN = 64
EMBRACE = 64
BB = 2048

_K0 = 0
_K1 = 42
_K2 = _K0 ^ _K1 ^ 0x1BD11BDA
_KS = (_K0, _K1, _K2)
_ROT = ((13, 15, 26, 6), (17, 29, 16, 24))


def _i32(v):
    v &= 0xFFFFFFFF
    return jnp.int32(v - 0x100000000 if v >= 0x80000000 else v)


def _threefry_bits(x1_keyed):
    x1 = x1_keyed
    x0 = None
    for i in range(5):
        for r in _ROT[i % 2]:
            x0 = x1 if x0 is None else x0 + x1
            x1 = (x1 << r) | jax.lax.shift_right_logical(x1, 32 - r)
            x1 = x1 ^ x0
        x0 = x0 + _i32(_KS[(i + 1) % 3])
        x1 = x1 + _i32(_KS[(i + 2) % 3] + i + 1)
    return x0 ^ x1


HALF = BATCH // 2


def _fuse_kernel(x_ref, w_ref, b_ref, o_ref):
    docked = []
    for m in range(N_MOD):
        xm = x_ref[m].reshape(2 * BB, D_IN)
        d = jnp.dot(xm, w_ref[m], preferred_element_type=jnp.float32)
        docked.append(jnp.maximum(d + b_ref[m][None, :], 0.0))

    r0 = pl.program_id(0) * BB
    rows = jax.lax.broadcasted_iota(jnp.int32, (BB, 128), 0) + r0
    lanes = jax.lax.broadcasted_iota(jnp.int32, (BB, 128), 1)
    base = (
        rows * (EMBRACE * N_MOD)
        + jax.lax.shift_right_logical(lanes, 6) * (HALF * EMBRACE * N_MOD)
        + (lanes & 63) * N_MOD
        + _i32(_K1)
    )

    best = jax.lax.shift_right_logical(_threefry_bits(base), 9)
    idx = jnp.zeros((BB, 128), jnp.int32)
    for m in range(1, N_MOD):
        v = jax.lax.shift_right_logical(_threefry_bits(base + m), 9)
        take = v > best
        idx = jnp.where(take, m, idx)
        best = jnp.maximum(v, best)

    for half, idx_h in ((0, idx[:, :EMBRACE]), (1, idx[:, EMBRACE:])):
        d = [docked[m][half * BB:(half + 1) * BB] for m in range(N_MOD)]
        res = jnp.where(
            idx_h < 2,
            jnp.where(idx_h == 0, d[0], d[1]),
            jnp.where(idx_h == 2, d[2], d[3]),
        )
        o_ref[half] = res


@jax.jit
def kernel(outputs1, outputs2, available, W, b):
    del outputs2, available
    grid = (HALF // BB,)
    o1 = outputs1.reshape(N_MOD, 2, HALF, D_IN)
    out = pl.pallas_call(
        _fuse_kernel,
        grid=grid,
        in_specs=[
            pl.BlockSpec((N_MOD, 2, BB, D_IN), lambda i: (0, 0, i, 0)),
            pl.BlockSpec((N_MOD, D_IN, EMBRACE), lambda i: (0, 0, 0)),
            pl.BlockSpec((N_MOD, EMBRACE), lambda i: (0, 0)),
        ],
        out_specs=pl.BlockSpec((2, BB, EMBRACE), lambda i: (0, i, 0)),
        out_shape=jax.ShapeDtypeStruct((2, HALF, EMBRACE), jnp.float32),
    )(o1, W, b)
    return out.reshape(BATCH, EMBRACE)


# speedup vs baseline: 1.5563x; 1.0745x over previous
import jax
import jax.numpy as jnp
from jax.experimental import pallas as pl

N_MOD = 4
BATCH = 16384
D_IN = 64
EMBRACE = 64
BB = 2048
B2 = 2 * BB

_K0 = 0
_K1 = 42
_K2 = _K0 ^ _K1 ^ 0x1BD11BDA
_KS = (_K0, _K1, _K2)
_ROT = ((13, 15, 26, 6), (17, 29, 16, 24))


def _i32(v):
    v &= 0xFFFFFFFF
    return jnp.int32(v - 0x100000000 if v >= 0x80000000 else v)


def _threefry_bits(x1_keyed):
    x1 = x1_keyed
    x0 = None
    for i in range(5):
        for r in _ROT[i % 2]:
            x0 = x1 if x0 is None else x0 + x1
            x1 = (x1 << r) | jax.lax.shift_right_logical(x1, 32 - r)
            x1 = x1 ^ x0
        x0 = x0 + _i32(_KS[(i + 1) % 3])
        x1 = x1 + _i32(_KS[(i + 2) % 3] + i + 1)
    return x0 ^ x1


def _fuse_kernel(x_ref, w_ref, b_ref, o_ref):
    docked = []
    for m in range(N_MOD):
        d = jnp.dot(x_ref[m], w_ref[m], preferred_element_type=jnp.float32)
        docked.append(jnp.maximum(d + b_ref[m][None, :], 0.0))

    r0 = pl.program_id(0) * B2
    rows = jax.lax.broadcasted_iota(jnp.int32, (BB, 128), 0) + r0
    lanes = jax.lax.broadcasted_iota(jnp.int32, (BB, 128), 1)
    base = (
        rows * (EMBRACE * N_MOD)
        + jax.lax.shift_right_logical(lanes, 6) * (BB * EMBRACE * N_MOD)
        + (lanes & 63) * N_MOD
        + _i32(_K1)
    )

    best = jax.lax.shift_right_logical(_threefry_bits(base), 9)
    idx = jnp.zeros((BB, 128), jnp.int32)
    for m in range(1, N_MOD):
        v = jax.lax.shift_right_logical(_threefry_bits(base + m), 9)
        take = v > best
        idx = jnp.where(take, m, idx)
        best = jnp.maximum(v, best)

    for half, idx_h in ((0, idx[:, :EMBRACE]), (1, idx[:, EMBRACE:])):
        d = [docked[m][half * BB:(half + 1) * BB] for m in range(N_MOD)]
        res = jnp.where(
            idx_h < 2,
            jnp.where(idx_h == 0, d[0], d[1]),
            jnp.where(idx_h == 2, d[2], d[3]),
        )
        o_ref[half * BB:(half + 1) * BB, :] = res


@jax.jit
def kernel(outputs1, outputs2, available, W, b):
    del outputs2, available
    grid = (BATCH // B2,)
    return pl.pallas_call(
        _fuse_kernel,
        grid=grid,
        in_specs=[
            pl.BlockSpec((N_MOD, B2, D_IN), lambda i: (0, i, 0)),
            pl.BlockSpec((N_MOD, D_IN, EMBRACE), lambda i: (0, 0, 0)),
            pl.BlockSpec((N_MOD, EMBRACE), lambda i: (0, 0)),
        ],
        out_specs=pl.BlockSpec((B2, EMBRACE), lambda i: (i, 0)),
        out_shape=jax.ShapeDtypeStruct((BATCH, EMBRACE), jnp.float32),
    )(outputs1, W, b)


# speedup vs baseline: 2.3562x; 1.5140x over previous
import jax
import jax.numpy as jnp
from jax.experimental import pallas as pl

N_MOD = 4
BATCH = 16384
D_IN = 64
EMBRACE = 64
BB = 4096

_K0 = 0
_K1 = 42
_K2 = _K0 ^ _K1 ^ 0x1BD11BDA
_KS = (_K0, _K1, _K2)
_ROT = ((13, 15, 26, 6), (17, 29, 16, 24))


def _i32(v):
    v &= 0xFFFFFFFF
    return jnp.int32(v - 0x100000000 if v >= 0x80000000 else v)


def _threefry_bits(x1_keyed):
    x1 = x1_keyed
    x0 = None
    for i in range(5):
        for r in _ROT[i % 2]:
            x0 = x1 if x0 is None else x0 + x1
            x1 = (x1 << r) | jax.lax.shift_right_logical(x1, 32 - r)
            x1 = x1 ^ x0
        x0 = x0 + _i32(_KS[(i + 1) % 3])
        x1 = x1 + _i32(_KS[(i + 2) % 3] + i + 1)
    return x0 ^ x1


def _fuse_kernel(x_ref, w_ref, b_ref, o_ref):
    docked = []
    for m in range(N_MOD):
        d = jax.lax.dot_general(
            w_ref[m], x_ref[m],
            dimension_numbers=(((0,), (0,)), ((), ())),
            preferred_element_type=jnp.float32,
        )
        docked.append(jnp.maximum(d + b_ref[m][:, None], 0.0))

    c0 = pl.program_id(0) * BB
    feats = jax.lax.broadcasted_iota(jnp.int32, (EMBRACE, BB), 0)
    cols = jax.lax.broadcasted_iota(jnp.int32, (EMBRACE, BB), 1) + c0
    base = cols * (EMBRACE * N_MOD) + feats * N_MOD + _i32(_K1)

    best = jax.lax.shift_right_logical(_threefry_bits(base), 9)
    idx = jnp.zeros((EMBRACE, BB), jnp.int32)
    for m in range(1, N_MOD):
        v = jax.lax.shift_right_logical(_threefry_bits(base + m), 9)
        take = v > best
        idx = jnp.where(take, m, idx)
        best = jnp.maximum(v, best)

    o_ref[...] = jnp.where(
        idx < 2,
        jnp.where(idx == 0, docked[0], docked[1]),
        jnp.where(idx == 2, docked[2], docked[3]),
    )


@jax.jit
def kernel(outputs1, outputs2, available, W, b):
    del outputs2, available
    x_t = jnp.transpose(outputs1, (0, 2, 1))
    out_t = pl.pallas_call(
        _fuse_kernel,
        grid=(BATCH // BB,),
        in_specs=[
            pl.BlockSpec((N_MOD, D_IN, BB), lambda i: (0, 0, i)),
            pl.BlockSpec((N_MOD, D_IN, EMBRACE), lambda i: (0, 0, 0)),
            pl.BlockSpec((N_MOD, EMBRACE), lambda i: (0, 0)),
        ],
        out_specs=pl.BlockSpec((EMBRACE, BB), lambda i: (0, i)),
        out_shape=jax.ShapeDtypeStruct((EMBRACE, BATCH), jnp.float32),
    )(x_t, W, b)
    return out_t.T


# speedup vs baseline: 2.3717x; 1.0066x over previous
import jax
import jax.numpy as jnp
from jax.experimental import pallas as pl

N_MOD = 4
BATCH = 16384
D_IN = 64
EMBRACE = 64
BB = 4096

_K0 = 0
_K1 = 42
_K2 = _K0 ^ _K1 ^ 0x1BD11BDA
_KS = (_K0, _K1, _K2)
_ROT = ((13, 15, 26, 6), (17, 29, 16, 24))


def _i32(v):
    v &= 0xFFFFFFFF
    return jnp.int32(v - 0x100000000 if v >= 0x80000000 else v)


def _threefry_bits(x1_keyed):
    x1 = x1_keyed
    x0 = None
    for i in range(5):
        for r in _ROT[i % 2]:
            x0 = x1 if x0 is None else x0 + x1
            x1 = (x1 << r) | jax.lax.shift_right_logical(x1, 32 - r)
            x1 = x1 ^ x0
        x0 = x0 + _i32(_KS[(i + 1) % 3])
        x1 = x1 + _i32(_KS[(i + 2) % 3] + i + 1)
    return x0 ^ x1


def _fuse_kernel(x_ref, w_ref, b_ref, o_ref):
    docked = []
    for m in range(N_MOD):
        d = jax.lax.dot_general(
            w_ref[m], x_ref[m],
            dimension_numbers=(((0,), (0,)), ((), ())),
            preferred_element_type=jnp.float32,
        )
        docked.append(jnp.maximum(d + b_ref[m][:, None], 0.0))

    c0 = pl.program_id(0) * BB
    feats = jax.lax.broadcasted_iota(jnp.int32, (EMBRACE, BB), 0)
    cols = jax.lax.broadcasted_iota(jnp.int32, (EMBRACE, BB), 1) + c0
    base = cols * (EMBRACE * N_MOD) + feats * N_MOD + _i32(_K1)

    best = jax.lax.shift_right_logical(_threefry_bits(base), 9)
    res = docked[0]
    for m in range(1, N_MOD):
        v = jax.lax.shift_right_logical(_threefry_bits(base + m), 9)
        take = v > best
        res = jnp.where(take, docked[m], res)
        best = jnp.maximum(v, best)

    o_ref[...] = res


@jax.jit
def kernel(outputs1, outputs2, available, W, b):
    del outputs2, available
    x_t = jnp.transpose(outputs1, (0, 2, 1))
    out_t = pl.pallas_call(
        _fuse_kernel,
        grid=(BATCH // BB,),
        in_specs=[
            pl.BlockSpec((N_MOD, D_IN, BB), lambda i: (0, 0, i)),
            pl.BlockSpec((N_MOD, D_IN, EMBRACE), lambda i: (0, 0, 0)),
            pl.BlockSpec((N_MOD, EMBRACE), lambda i: (0, 0)),
        ],
        out_specs=pl.BlockSpec((EMBRACE, BB), lambda i: (0, i)),
        out_shape=jax.ShapeDtypeStruct((EMBRACE, BATCH), jnp.float32),
    )(x_t, W, b)
    return out_t.T


# speedup vs baseline: 2.3857x; 1.0059x over previous
import jax
import jax.numpy as jnp
from jax.experimental import pallas as pl

N_MOD = 4
BATCH = 16384
D_IN = 64
EMBRACE = 64
BB = 2048

_K0 = 0
_K1 = 42
_K2 = _K0 ^ _K1 ^ 0x1BD11BDA
_KS = (_K0, _K1, _K2)
_ROT = ((13, 15, 26, 6), (17, 29, 16, 24))


def _i32(v):
    v &= 0xFFFFFFFF
    return jnp.int32(v - 0x100000000 if v >= 0x80000000 else v)


def _threefry_bits(x1_keyed):
    x1 = x1_keyed
    x0 = None
    for i in range(5):
        for r in _ROT[i % 2]:
            x0 = x1 if x0 is None else x0 + x1
            x1 = (x1 << r) | jax.lax.shift_right_logical(x1, 32 - r)
            x1 = x1 ^ x0
        x0 = x0 + _i32(_KS[(i + 1) % 3])
        x1 = x1 + _i32(_KS[(i + 2) % 3] + i + 1)
    return x0 ^ x1


def _fuse_kernel(x_ref, w_ref, b_ref, o_ref):
    docked = []
    for m in range(N_MOD):
        d = jax.lax.dot_general(
            w_ref[m], x_ref[m],
            dimension_numbers=(((0,), (0,)), ((), ())),
            preferred_element_type=jnp.float32,
        )
        docked.append(jnp.maximum(d + b_ref[m][:, None], 0.0))

    c0 = pl.program_id(0) * BB
    feats = jax.lax.broadcasted_iota(jnp.int32, (EMBRACE, BB), 0)
    cols = jax.lax.broadcasted_iota(jnp.int32, (EMBRACE, BB), 1) + c0
    base = cols * (EMBRACE * N_MOD) + feats * N_MOD + _i32(_K1)

    best = jax.lax.shift_right_logical(_threefry_bits(base), 9)
    res = docked[0]
    for m in range(1, N_MOD):
        v = jax.lax.shift_right_logical(_threefry_bits(base + m), 9)
        take = v > best
        res = jnp.where(take, docked[m], res)
        best = jnp.maximum(v, best)

    o_ref[...] = res


@jax.jit
def kernel(outputs1, outputs2, available, W, b):
    del outputs2, available
    x_t = jnp.transpose(outputs1, (0, 2, 1))
    out_t = pl.pallas_call(
        _fuse_kernel,
        grid=(BATCH // BB,),
        in_specs=[
            pl.BlockSpec((N_MOD, D_IN, BB), lambda i: (0, 0, i)),
            pl.BlockSpec((N_MOD, D_IN, EMBRACE), lambda i: (0, 0, 0)),
            pl.BlockSpec((N_MOD, EMBRACE), lambda i: (0, 0)),
        ],
        out_specs=pl.BlockSpec((EMBRACE, BB), lambda i: (0, i)),
        out_shape=jax.ShapeDtypeStruct((EMBRACE, BATCH), jnp.float32),
    )(x_t, W, b)
    return out_t.T


# speedup vs baseline: 2.3880x; 1.0009x over previous
import jax
import jax.numpy as jnp
from jax.experimental import pallas as pl

N_MOD = 4
BATCH = 16384
D_IN = 64
EMBRACE = 64
BB = 1024

_K0 = 0
_K1 = 42
_K2 = _K0 ^ _K1 ^ 0x1BD11BDA
_KS = (_K0, _K1, _K2)
_ROT = ((13, 15, 26, 6), (17, 29, 16, 24))


def _i32(v):
    v &= 0xFFFFFFFF
    return jnp.int32(v - 0x100000000 if v >= 0x80000000 else v)


def _threefry_bits(x1_keyed):
    x1 = x1_keyed
    x0 = None
    for i in range(5):
        for r in _ROT[i % 2]:
            x0 = x1 if x0 is None else x0 + x1
            x1 = (x1 << r) | jax.lax.shift_right_logical(x1, 32 - r)
            x1 = x1 ^ x0
        x0 = x0 + _i32(_KS[(i + 1) % 3])
        x1 = x1 + _i32(_KS[(i + 2) % 3] + i + 1)
    return x0 ^ x1


def _fuse_kernel(x_ref, w_ref, b_ref, o_ref):
    docked = []
    for m in range(N_MOD):
        d = jax.lax.dot_general(
            w_ref[m], x_ref[m],
            dimension_numbers=(((0,), (0,)), ((), ())),
            preferred_element_type=jnp.float32,
        )
        docked.append(jnp.maximum(d + b_ref[m][:, None], 0.0))

    c0 = pl.program_id(0) * BB
    feats = jax.lax.broadcasted_iota(jnp.int32, (EMBRACE, BB), 0)
    cols = jax.lax.broadcasted_iota(jnp.int32, (EMBRACE, BB), 1) + c0
    base = cols * (EMBRACE * N_MOD) + feats * N_MOD + _i32(_K1)

    best = jax.lax.shift_right_logical(_threefry_bits(base), 9)
    res = docked[0]
    for m in range(1, N_MOD):
        v = jax.lax.shift_right_logical(_threefry_bits(base + m), 9)
        take = v > best
        res = jnp.where(take, docked[m], res)
        best = jnp.maximum(v, best)

    o_ref[...] = res


@jax.jit
def kernel(outputs1, outputs2, available, W, b):
    del outputs2, available
    x_t = jnp.transpose(outputs1, (0, 2, 1))
    out_t = pl.pallas_call(
        _fuse_kernel,
        grid=(BATCH // BB,),
        in_specs=[
            pl.BlockSpec((N_MOD, D_IN, BB), lambda i: (0, 0, i)),
            pl.BlockSpec((N_MOD, D_IN, EMBRACE), lambda i: (0, 0, 0)),
            pl.BlockSpec((N_MOD, EMBRACE), lambda i: (0, 0)),
        ],
        out_specs=pl.BlockSpec((EMBRACE, BB), lambda i: (0, i)),
        out_shape=jax.ShapeDtypeStruct((EMBRACE, BATCH), jnp.float32),
    )(x_t, W, b)
    return out_t.T
